# baseline (speedup 1.0000x reference)
import functools

import jax
import jax.numpy as jnp
from jax import lax
from jax.experimental import pallas as pl
from jax.experimental.pallas import tpu as pltpu
from jax.experimental.pallas import tpu_sc as plsc

VOCAB = 1000000
SEQ_LEN = 200
EMBED_DIM = 32
BATCH = 4096

NUM_CORES = 2
NUM_SUBCORES = 16
NUM_WORKERS = NUM_CORES * NUM_SUBCORES

LANES = 16
BB = 128
HB = BB // 2
NBLK = BATCH // BB
HH_N = BB // LANES
DB = 8
NG = EMBED_DIM // DB
GRP = 4
NGRP = NBLK // GRP

L_BASE = SEQ_LEN // NUM_WORKERS
L_EXTRA = SEQ_LEN % NUM_WORKERS
L_MAX = L_BASE + 1
SEQ_PAD = 208

OUT_ROWS = SEQ_LEN * NG * NBLK


def _body(idx_hbm, tok_hbm, pos_hbm, out_hbm,
          idx_v, r0_v, r1_v, o0_v, o1_v, pos_v, psplat_v,
          gsem0, gsem1, osem0, osem1):
    wid = lax.axis_index("s") * NUM_CORES + lax.axis_index("c")

    lo = wid * L_BASE + jnp.minimum(wid, L_EXTRA)
    cnt = L_BASE + jnp.where(wid < L_EXTRA, 1, 0)
    n_groups = cnt * NGRP

    pltpu.sync_copy(idx_hbm.at[pl.ds(lo, L_MAX)], idx_v)
    pltpu.sync_copy(pos_hbm.at[pl.ds(lo, DB)], pos_v)

    iota = lax.iota(jnp.int32, LANES)
    zeros16 = jnp.zeros((LANES,), jnp.int32)
    def diag_cols(t, k):
        col = lax.bitwise_and(iota + k, LANES - 1) + 16 * t
        return (
            col,
            lax.shift_right_logical(col, 3),
            lax.bitwise_and(col, DB - 1),
        )

    r_bufs = (r0_v, r1_v)
    o_bufs = (o0_v, o1_v)
    gsems = (gsem0, gsem1)
    osems = (osem0, osem1)

    def fire_group(gg, b):
        l_off = lax.shift_right_logical(gg, 3)
        g = lax.bitwise_and(gg, NGRP - 1)
        for j in range(GRP):
            idx_row = idx_v.at[l_off, g * GRP + j]
            for h in range(2):
                pltpu.async_copy(
                    tok_hbm.at[idx_row.at[pl.ds(h * HB, HB)]],
                    r_bufs[b].at[j, pl.ds(h * HB, HB)],
                    gsems[b],
                )

    def drain_gathers(b):
        for j in range(GRP):
            for h in range(2):
                pltpu.make_async_copy(
                    tok_hbm.at[pl.ds(0, HB)],
                    r_bufs[b].at[j, pl.ds(h * HB, HB)],
                    gsems[b],
                ).wait()

    def drain_outs(b):
        for g_ in range(NG):
            pltpu.make_async_copy(
                out_hbm.at[pl.ds(0, GRP)], o_bufs[b].at[g_], osems[b]
            ).wait()

    fire_group(0, 0)
    fire_group(1, 1)

    def pair_body(t, carry):
        for b in range(2):
            gg = 2 * t + b
            l_off = lax.shift_right_logical(gg, 3)
            g = lax.bitwise_and(gg, NGRP - 1)
            l = lo + l_off
            rbuf, obuf = r_bufs[b], o_bufs[b]
            drain_gathers(b)

            @pl.when(t >= 1)
            def _():
                drain_outs(b)

            if b == 0:
                @pl.when(g == 0)
                def _():
                    l_vec = zeros16 + l_off
                    for t in range(2):
                        for k in range(LANES):
                            col, _, _ = diag_cols(t, k)
                            psplat_v[16 * t + k, :] = plsc.load_gather(
                                pos_v, [l_vec, col]
                            )

            @plsc.parallel_loop(0, GRP * HH_N, unroll=1)
            def _(u):
                cc = lax.shift_right_logical(u, 3)
                hh = lax.bitwise_and(u, HH_N - 1)
                rblk = rbuf.at[cc]
                row_vec = iota + hh * LANES
                cc_vec = zeros16 + cc
                for t in range(2):
                    for k in range(LANES):
                        col, g_v, dd_v = diag_cols(t, k)
                        vals = plsc.load_gather(rblk, [row_vec, col])
                        vals = vals + psplat_v[16 * t + k, :]
                        plsc.store_scatter(
                            obuf, [g_v, cc_vec, dd_v, row_vec], vals
                        )

            for g_ in range(NG):
                row0 = (l * NG + g_) * NBLK + g * GRP
                pltpu.async_copy(
                    obuf.at[g_], out_hbm.at[pl.ds(row0, GRP)], osems[b]
                )

            @pl.when(gg + 2 < n_groups)
            def _():
                fire_group(gg + 2, b)

        return carry

    lax.fori_loop(0, cnt * (NGRP // 2), pair_body, 0)
    drain_outs(0)
    drain_outs(1)


_mesh = plsc.VectorSubcoreMesh(core_axis_name="c", subcore_axis_name="s")

_sc_call = functools.partial(
    pl.kernel,
    out_type=jax.ShapeDtypeStruct((OUT_ROWS, DB, BB), jnp.float32),
    mesh=_mesh,
    scratch_types=[
        pltpu.VMEM((L_MAX, NBLK, BB), jnp.int32),
        pltpu.VMEM((GRP, BB, EMBED_DIM), jnp.float32),
        pltpu.VMEM((GRP, BB, EMBED_DIM), jnp.float32),
        pltpu.VMEM((NG, GRP, DB, BB), jnp.float32),
        pltpu.VMEM((NG, GRP, DB, BB), jnp.float32),
        pltpu.VMEM((DB, EMBED_DIM), jnp.float32),
        pltpu.VMEM((EMBED_DIM, LANES), jnp.float32),
        pltpu.SemaphoreType.DMA,
        pltpu.SemaphoreType.DMA,
        pltpu.SemaphoreType.DMA,
        pltpu.SemaphoreType.DMA,
    ],
    compiler_params=pltpu.CompilerParams(
        use_tc_tiling_on_sc=False, needs_layout_passes=False
    ),
)


@jax.jit
def kernel(inputs, token_table, pos_table):
    idx = inputs.astype(jnp.int32).T.reshape(SEQ_LEN, NBLK, BB)
    idx = jnp.pad(idx, ((0, SEQ_PAD - SEQ_LEN), (0, 0), (0, 0)))
    pos = jnp.pad(pos_table, ((0, SEQ_PAD - SEQ_LEN), (0, 0)))
    o3 = _sc_call(_body)(idx, token_table, pos)
    o5 = o3.reshape(SEQ_LEN, NG, NBLK, DB, BB)
    out = o5.transpose(0, 1, 3, 2, 4).reshape(SEQ_LEN, EMBED_DIM, BATCH)
    return out.transpose(2, 0, 1)
